# baseline (speedup 1.0000x reference)
import functools

import jax
import jax.numpy as jnp
from jax import lax
from jax.experimental import pallas as pl
from jax.experimental.pallas import tpu as pltpu
from jax.experimental.pallas import tpu_sc as plsc

_NC = 2
_NS = 16

_B1 = 20


def _sc_gather(hours_flat, pe, b_start, b2, T, F):
    mesh = plsc.VectorSubcoreMesh(
        core_axis_name="c", subcore_axis_name="s", num_cores=_NC, num_subcores=_NS
    )

    @functools.partial(
        pl.kernel,
        out_type=jax.ShapeDtypeStruct((b2 * T, F), jnp.float32),
        mesh=mesh,
        scratch_types=[
            pltpu.VMEM((T,), jnp.int32),
            pltpu.VMEM((T, F), jnp.float32),
            pltpu.SemaphoreType.DMA,
        ],
        compiler_params=pltpu.CompilerParams(use_tc_tiling_on_sc=False),
    )
    def gather_kernel(hours_hbm, pe_hbm, pos_hbm, idx_v, rows_v, sem):
        wid = lax.axis_index("s") * _NC + lax.axis_index("c")

        @pl.when(wid < b2)
        def _():
            pltpu.sync_copy(hours_hbm.at[pl.ds((b_start + wid) * T, T)], idx_v)
            pltpu.async_copy(pe_hbm.at[idx_v], rows_v, sem).wait()
            pltpu.sync_copy(rows_v, pos_hbm.at[pl.ds(wid * T, T)])

    return gather_kernel(hours_flat, pe)


def _tc_broadcast_a(hours, pe, B, N, T, F, b1):
    V = pe.shape[0]

    def body(hours_smem, pe_ref, out_ref):
        b = pl.program_id(0)
        for t in range(T):
            row = pe_ref[hours_smem[b, t], :]
            out_ref[0, t] = jnp.broadcast_to(row[:, None], (F, N))

    grid_spec = pltpu.PrefetchScalarGridSpec(
        num_scalar_prefetch=1,
        grid=(b1,),
        in_specs=[pl.BlockSpec((V, F), lambda b, hrs: (0, 0))],
        out_specs=pl.BlockSpec((1, T, F, N), lambda b, hrs: (b, 0, 0, 0)),
    )
    return pl.pallas_call(
        body,
        grid_spec=grid_spec,
        out_shape=jax.ShapeDtypeStruct((B, T, F, N), jnp.float32),
    )(hours, pe)


def _tc_broadcast_b(pos_flat, out_a, B, N, T, F, b_start, b2):
    def body(pos_ref, out_alias_ref, out_ref):
        i = pl.program_id(0)
        for t2 in range(T // 2):
            pair = pos_ref[pl.ds(i * T * F + t2 * 2 * F, 2 * F)]
            out_ref[0, 2 * t2] = jnp.broadcast_to(pair[:F, None], (F, N))
            out_ref[0, 2 * t2 + 1] = jnp.broadcast_to(pair[F:, None], (F, N))

    return pl.pallas_call(
        body,
        grid=(b2,),
        in_specs=[
            pl.BlockSpec((b2 * T * F,), lambda i: (0,)),
            pl.BlockSpec(memory_space=pltpu.MemorySpace.HBM),
        ],
        out_specs=pl.BlockSpec((1, T, F, N), lambda i: (b_start + i, 0, 0, 0)),
        out_shape=jax.ShapeDtypeStruct((B, T, F, N), jnp.float32),
        input_output_aliases={1: 0},
    )(pos_flat, out_a)


def kernel(x, hours, pe):
    B, N, T, F = x.shape
    hours = hours.astype(jnp.int32)
    b1 = min(_B1, B)
    b2 = B - b1
    if b2 == 0:
        out_t = _tc_broadcast_a(hours, pe, B, N, T, F, b1)
        return jnp.transpose(out_t, (0, 3, 1, 2))
    pos = _sc_gather(hours.reshape(B * T), pe, b1, b2, T, F)
    out_a = _tc_broadcast_a(hours, pe, B, N, T, F, b1)
    out_t = _tc_broadcast_b(pos.reshape(b2 * T * F), out_a, B, N, T, F, b1, b2)
    return jnp.transpose(out_t, (0, 3, 1, 2))
